# baseline (speedup 1.0000x reference)
import functools

import jax
import jax.numpy as jnp
from jax import lax
from jax.experimental import pallas as pl
from jax.experimental.pallas import tpu as pltpu
from jax.experimental.pallas import tpu_sc as plsc

N_EXPERTS = 64
TOPK = 8
N_ROWS = 16384
BLOCK_ROWS = 1024
CHUNK_ROWS = 256

NC = 2
NS = 16
NW = NC * NS
ROWS_PER_W = N_ROWS // NW
GROUPS = ROWS_PER_W // 16


def _matmul_kernel(x_ref, w_ref, out_ref):
    for c in range(BLOCK_ROWS // CHUNK_ROWS):
        rows = pl.ds(c * CHUNK_ROWS, CHUNK_ROWS)
        out_ref[:, rows] = jax.lax.dot_general(
            w_ref[...],
            x_ref[rows, :],
            dimension_numbers=(((1,), (1,)), ((), ())),
            preferred_element_type=jnp.float32,
        )


def _tc_logits_t(x, gate_w):
    return pl.pallas_call(
        _matmul_kernel,
        grid=(N_ROWS // BLOCK_ROWS,),
        in_specs=[
            pl.BlockSpec((BLOCK_ROWS, x.shape[1]), lambda i: (i, 0)),
            pl.BlockSpec((N_EXPERTS, x.shape[1]), lambda i: (0, 0)),
        ],
        out_specs=pl.BlockSpec((N_EXPERTS, BLOCK_ROWS), lambda i: (0, i)),
        out_shape=jax.ShapeDtypeStruct((N_EXPERTS, N_ROWS), jnp.float32),
    )(x, gate_w)


@functools.partial(
    pl.kernel,
    mesh=plsc.VectorSubcoreMesh(core_axis_name="c", subcore_axis_name="s"),
    out_type=[
        jax.ShapeDtypeStruct((TOPK, N_ROWS), jnp.float32),
        jax.ShapeDtypeStruct((TOPK, N_ROWS), jnp.int32),
    ],
    scratch_types=[
        pltpu.VMEM((N_EXPERTS, ROWS_PER_W), jnp.float32),
        pltpu.VMEM((TOPK, ROWS_PER_W), jnp.float32),
        pltpu.VMEM((TOPK, ROWS_PER_W), jnp.int32),
    ],
)
def _sc_topk(lt_hbm, wout_hbm, iout_hbm, lt_v, w_v, i_v):
    wid = lax.axis_index("s") * NC + lax.axis_index("c")
    base = wid * ROWS_PER_W
    pltpu.sync_copy(lt_hbm.at[:, pl.ds(base, ROWS_PER_W)], lt_v)

    neg_inf = jnp.full((16,), -jnp.inf, jnp.float32)

    def group_body(g, carry):
        cols = pl.ds(g * 16, 16)
        vals = []
        idxs = []
        idx_prev = jnp.full((16,), -1, jnp.int32)
        for _k in range(TOPK):
            m = neg_inf
            idx = jnp.full((16,), 0, jnp.int32)
            for e in range(N_EXPERTS):
                e_vec = jnp.full((16,), e, jnp.int32)
                v = lt_v[e, cols]
                dead = idx_prev == e_vec
                v = jnp.where(dead, neg_inf, v)
                lt_v[e, cols] = v
                c2 = v > m
                m = jnp.where(c2, v, m)
                idx = jnp.where(c2, e_vec, idx)
            vals.append(m)
            idxs.append(idx)
            idx_prev = idx
        es = [jnp.exp(v - vals[0]) for v in vals]
        s = es[0]
        for e_ in es[1:]:
            s = s + e_
        for k_ in range(TOPK):
            w_v[k_, cols] = es[k_] / s
            i_v[k_, cols] = idxs[k_]
        return carry

    lax.fori_loop(0, GROUPS, group_body, 0)

    pltpu.sync_copy(w_v, wout_hbm.at[:, pl.ds(base, ROWS_PER_W)])
    pltpu.sync_copy(i_v, iout_hbm.at[:, pl.ds(base, ROWS_PER_W)])


@jax.jit
def kernel(x, gate_w):
    logits_t = _tc_logits_t(x, gate_w)
    w_t, i_t = _sc_topk(logits_t)
    return (w_t.T, i_t.T)


# speedup vs baseline: 1.3388x; 1.3388x over previous
import functools

import jax
import jax.numpy as jnp
from jax.experimental import pallas as pl

N_EXPERTS = 64
TOPK = 8
BLOCK_ROWS = 1024
CHUNK_ROWS = 256


def _topk_softmax_t(logits_t):
    iota = jax.lax.broadcasted_iota(jnp.int32, logits_t.shape, 0)
    l = logits_t
    vals = []
    idxs = []
    for _ in range(TOPK):
        m = jnp.max(l, axis=0, keepdims=True)
        idx = jnp.min(
            jnp.where(l == m, iota, N_EXPERTS), axis=0, keepdims=True
        )
        vals.append(m)
        idxs.append(idx)
        l = jnp.where(iota == idx, -jnp.inf, l)

    v = jnp.concatenate(vals, axis=0)
    e = jnp.exp(v - vals[0])
    w = e / jnp.sum(e, axis=0, keepdims=True)
    return w, jnp.concatenate(idxs, axis=0)


def _router_kernel(x_ref, w_ref, wout_ref, iout_ref):
    for c in range(BLOCK_ROWS // CHUNK_ROWS):
        rows = pl.ds(c * CHUNK_ROWS, CHUNK_ROWS)
        logits_t = jax.lax.dot_general(
            w_ref[...],
            x_ref[rows, :],
            dimension_numbers=(((1,), (1,)), ((), ())),
            preferred_element_type=jnp.float32,
        )
        w, i = _topk_softmax_t(logits_t)
        wout_ref[rows, :] = w.T
        iout_ref[rows, :] = i.T


@jax.jit
def kernel(x, gate_w):
    n_rows = x.shape[0]
    grid = (n_rows // BLOCK_ROWS,)
    wout, iout = pl.pallas_call(
        _router_kernel,
        grid=grid,
        in_specs=[
            pl.BlockSpec((BLOCK_ROWS, x.shape[1]), lambda i: (i, 0)),
            pl.BlockSpec((N_EXPERTS, x.shape[1]), lambda i: (0, 0)),
        ],
        out_specs=[
            pl.BlockSpec((BLOCK_ROWS, TOPK), lambda i: (i, 0)),
            pl.BlockSpec((BLOCK_ROWS, TOPK), lambda i: (i, 0)),
        ],
        out_shape=[
            jax.ShapeDtypeStruct((n_rows, TOPK), jnp.float32),
            jax.ShapeDtypeStruct((n_rows, TOPK), jnp.int32),
        ],
    )(x, gate_w)
    return (wout, iout)
